# Initial kernel scaffold; baseline (speedup 1.0000x reference)
import jax
import jax.numpy as jnp
from jax.experimental import pallas as pl


def kernel(x, edge_index, edge_attr, batch, params):
    raise NotImplementedError("write your pallas kernel here")


# speedup vs baseline: 2.9353x; 2.9353x over previous
import functools

import jax
import jax.numpy as jnp
from jax import lax
from jax.experimental import pallas as pl
from jax.experimental.pallas import tpu as pltpu
from jax.experimental.pallas import tpu_sc as plsc

N = 10000
E = 320000
D = 128
DE = 16
G = 64

NC = 2
NS = 16
CH = 80
EPW = E // (NC * NS)
RPT = 624
RPT_LAST = N - 15 * RPT
F32 = jnp.float32


def _sc_body(h_hbm, eb_hbm, src_hbm, dst_hbm, zr_hbm, out_hbm,
             srcv, dstv, gv, ev, aggr, sem):
    c = lax.axis_index("c")
    s = lax.axis_index("s")
    wid = s * NC + c
    @pl.when(s < NS - 1)
    def _():
        pltpu.sync_copy(zr_hbm.at[pl.ds(0, RPT)], aggr.at[pl.ds(s * RPT, RPT)])

    @pl.when(s == NS - 1)
    def _():
        pltpu.sync_copy(zr_hbm, aggr.at[pl.ds(15 * RPT, RPT_LAST)])

    plsc.subcore_barrier()

    base = wid * EPW

    def chunk(i, carry):
        off = base + i * CH
        pltpu.sync_copy(src_hbm.at[pl.ds(off, CH)], srcv)
        pltpu.sync_copy(dst_hbm.at[pl.ds(off, CH)], dstv)
        cp = pltpu.async_copy(h_hbm.at[srcv], gv, sem)
        pltpu.sync_copy(eb_hbm.at[pl.ds(off, CH)], ev)
        cp.wait()

        def row(j, cc):
            for k in range(D // 16):
                sl = pl.ds(k * 16, 16)
                gv[j, sl] = jnp.maximum(gv[j, sl] + ev[j, sl], 0.0)
            return cc

        lax.fori_loop(0, CH, row, 0)
        pltpu.sync_copy(gv, aggr.at[dstv], add=True)
        return carry

    lax.fori_loop(0, EPW // CH, chunk, 0)
    plsc.subcore_barrier()

    @pl.when(s < NS - 1)
    def _():
        pltpu.sync_copy(aggr.at[pl.ds(s * RPT, RPT)],
                        out_hbm.at[pl.ds(c * N + s * RPT, RPT)])

    @pl.when(s == NS - 1)
    def _():
        pltpu.sync_copy(aggr.at[pl.ds(15 * RPT, RPT_LAST)],
                        out_hbm.at[pl.ds(c * N + 15 * RPT, RPT_LAST)])


_sc_aggr = pl.kernel(
    _sc_body,
    out_type=jax.ShapeDtypeStruct((2 * N, D), F32),
    mesh=plsc.VectorSubcoreMesh(core_axis_name="c", subcore_axis_name="s",
                                num_cores=NC, num_subcores=NS),
    scratch_types=[
        pltpu.VMEM((CH,), jnp.int32),
        pltpu.VMEM((CH,), jnp.int32),
        pltpu.VMEM((CH, D), F32),
        pltpu.VMEM((CH, D), F32),
        pltpu.VMEM_SHARED((N, D), F32),
        pltpu.SemaphoreType.DMA,
    ],
)


BE = 3200


def _edge_lin_body(ea_ref, w_ref, b_ref, o1, o2, o3):
    ea = ea_ref[...]
    for l, o in enumerate((o1, o2, o3)):
        o[...] = (jnp.dot(ea, w_ref[l]) + b_ref[l])


def _edge_lin(ea, w_stack, b_stack):
    eshape = jax.ShapeDtypeStruct((E, D), F32)
    return pl.pallas_call(
        _edge_lin_body,
        grid=(E // BE,),
        in_specs=[
            pl.BlockSpec((BE, DE), lambda i: (i, 0)),
            pl.BlockSpec((3, DE, D), lambda i: (0, 0, 0)),
            pl.BlockSpec((3, 1, D), lambda i: (0, 0, 0)),
        ],
        out_specs=[pl.BlockSpec((BE, D), lambda i: (i, 0))] * 3,
        out_shape=[eshape, eshape, eshape],
    )(ea, w_stack, b_stack)


BN = 2000


def _mlp_body(h_ref, p_ref, w1_ref, b1_ref, w2_ref, b2_ref, o_ref):
    y = h_ref[...] + p_ref[0] + p_ref[1]
    t = jax.nn.relu(jnp.dot(y, w1_ref[...]) + b1_ref[...])
    o_ref[...] = jax.nn.relu(
        jnp.dot(t, w2_ref[...]) + b2_ref[...])


def _mlp(h, parts, w1, b1, w2, b2):
    return pl.pallas_call(
        _mlp_body,
        grid=(N // BN,),
        in_specs=[
            pl.BlockSpec((BN, D), lambda i: (i, 0)),
            pl.BlockSpec((2, BN, D), lambda i: (0, i, 0)),
            pl.BlockSpec((D, D), lambda i: (0, 0)),
            pl.BlockSpec((1, D), lambda i: (0, 0)),
            pl.BlockSpec((D, D), lambda i: (0, 0)),
            pl.BlockSpec((1, D), lambda i: (0, 0)),
        ],
        out_specs=pl.BlockSpec((BN, D), lambda i: (i, 0)),
        out_shape=jax.ShapeDtypeStruct((N, D), F32),
    )(h, parts, w1, b1.reshape(1, D), w2, b2.reshape(1, D))


def _pool_body(h_ref, b_ref, wih_ref, bih_ref, bhh_ref, rw_ref, rb_ref, o_ref):
    h = h_ref[...]
    gids = lax.broadcasted_iota(jnp.int32, (N, G), 1)
    onehot = (b_ref[...] == gids).astype(F32)
    sums = lax.dot_general(onehot, h, (((0,), (0,)), ((), ())),
                           precision=lax.Precision.HIGHEST)
    cnts = jnp.sum(onehot, axis=0)
    pooled = sums / jnp.maximum(cnts, 1.0)[:, None]
    z = lax.dot_general(pooled, wih_ref[...], (((1,), (1,)), ((), ())))
    z = z + bih_ref[...] + bhh_ref[...]
    i_g = z[:, 0:D]
    g_g = z[:, 2 * D:3 * D]
    o_g = z[:, 3 * D:4 * D]
    cst = jax.nn.sigmoid(i_g) * jnp.tanh(g_g)
    hh = jax.nn.sigmoid(o_g) * jnp.tanh(cst)
    o_ref[...] = jnp.dot(hh, rw_ref[...]) + rb_ref[...]


def _pool_lstm(h, batch2d, wih, bih, bhh, rw, rb):
    full = lambda shape: pl.BlockSpec(shape, lambda: tuple(0 for _ in shape))
    return pl.pallas_call(
        _pool_body,
        in_specs=[
            full((N, D)), full((N, 1)), full((4 * D, D)),
            full((1, 4 * D)), full((1, 4 * D)), full((D, 1)), full((1, 1)),
        ],
        out_specs=full((G, 1)),
        out_shape=jax.ShapeDtypeStruct((G, 1), F32),
    )(h, batch2d, wih, bih.reshape(1, 4 * D), bhh.reshape(1, 4 * D),
      rw, rb.reshape(1, 1))


def kernel(x, edge_index, edge_attr, batch, params):
    p = params
    w_stack = jnp.stack([p['lin1_W'], p['lin2_W'], p['lin3_W']])
    b_stack = jnp.stack([p['lin1_b'], p['lin2_b'], p['lin3_b']])[:, None, :]
    eb1, eb2, eb3 = _edge_lin(edge_attr, w_stack, b_stack)
    zr = jnp.zeros((RPT_LAST, D), F32)
    src = edge_index[0]
    dst = edge_index[1]

    h = x
    for eb, wk in ((eb1, 'n1'), (eb2, 'n2'), (eb3, 'n3')):
        parts = _sc_aggr(h, eb, src, dst, zr)
        parts = parts.reshape(2, N, D)
        h = _mlp(h, parts, p[wk + '_W1'], p[wk + '_b1'],
                 p[wk + '_W2'], p[wk + '_b2'])

    out = _pool_lstm(h, batch.reshape(N, 1), p['Wih'], p['bih'], p['bhh'],
                     p['reg_W'], p['reg_b'])
    return out[:, 0]


# speedup vs baseline: 4.7998x; 1.6352x over previous
import functools

import jax
import jax.numpy as jnp
from jax import lax
from jax.experimental import pallas as pl
from jax.experimental.pallas import tpu as pltpu
from jax.experimental.pallas import tpu_sc as plsc

N = 10000
E = 320000
D = 128
DE = 16
G = 64

NC = 2
NS = 16
CH = 40
EPW = E // (NC * NS)
RPT = 624
RPT_LAST = N - 15 * RPT
F32 = jnp.float32


NCHUNK = EPW // CH


def _sc_body(h_hbm, eb_hbm, src_hbm, dst_hbm, zr_hbm, out_hbm,
             sv0, sv1, sv2, sv3, dv0, dv1, gv0, gv1, ev0, ev1, aggr,
             qs0, qs1, qs2, qs3, qd0, qd1, sg0, sg1, se0, se1):
    c = lax.axis_index("c")
    s = lax.axis_index("s")
    wid = s * NC + c
    base = wid * EPW
    svs = (sv0, sv1, sv2, sv3)
    qss = (qs0, qs1, qs2, qs3)

    def sfire(j_static_mod, j, q=None):
        pltpu.async_copy(src_hbm.at[pl.ds(base + j * CH, CH)],
                         svs[j_static_mod], qss[j_static_mod])

    def swait(j_static_mod, j):
        pltpu.make_async_copy(src_hbm.at[pl.ds(base + j * CH, CH)],
                              svs[j_static_mod], qss[j_static_mod]).wait()

    def dfire(dvb, qdb, j):
        pltpu.async_copy(dst_hbm.at[pl.ds(base + j * CH, CH)], dvb, qdb)

    def dwait(dvb, qdb, j):
        pltpu.make_async_copy(dst_hbm.at[pl.ds(base + j * CH, CH)], dvb,
                              qdb).wait()

    def gefire(j, svb, gvb, evb, sgb, seb):
        pltpu.async_copy(h_hbm.at[svb], gvb, sgb)
        pltpu.async_copy(eb_hbm.at[pl.ds(base + j * CH, CH)], evb, seb)

    def gewait(j, svb, gvb, evb, sgb, seb):
        pltpu.make_async_copy(h_hbm.at[svb], gvb, sgb).wait()
        pltpu.make_async_copy(eb_hbm.at[pl.ds(base + j * CH, CH)], evb,
                              seb).wait()

    def compute(gvb, evb):
        def row(r, cc):
            for k in range(D // 16):
                sl = pl.ds(k * 16, 16)
                gvb[r, sl] = jnp.maximum(gvb[r, sl] + evb[r, sl], 0.0)
            return cc
        lax.fori_loop(0, CH, row, 0)

    for m in range(4):
        sfire(m, m)
    dfire(dv0, qd0, 0)
    dfire(dv1, qd1, 1)
    swait(0, 0)
    swait(1, 1)
    gefire(0, sv0, gv0, ev0, sg0, se0)
    gefire(1, sv1, gv1, ev1, sg1, se1)

    @pl.when(s < NS - 1)
    def _():
        pltpu.sync_copy(zr_hbm.at[pl.ds(0, RPT)], aggr.at[pl.ds(s * RPT, RPT)])

    @pl.when(s == NS - 1)
    def _():
        pltpu.sync_copy(zr_hbm, aggr.at[pl.ds(15 * RPT, RPT_LAST)])

    plsc.subcore_barrier()

    def half(i, m0, dvb, qdb, gvb, evb, sgb, seb):
        m2 = (m0 + 2) % 4
        gewait(i, svs[m0], gvb, evb, sgb, seb)

        @pl.when(i + 4 < NCHUNK)
        def _():
            sfire(m0, i + 4)

        dwait(dvb, qdb, i)
        compute(gvb, evb)
        pltpu.sync_copy(gvb, aggr.at[dvb], add=True)

        @pl.when(i + 2 < NCHUNK)
        def _():
            dfire(dvb, qdb, i + 2)
            swait(m2, i + 2)
            gefire(i + 2, svs[m2], gvb, evb, sgb, seb)

    def quad(q, carry):
        i = 4 * q
        half(i, 0, dv0, qd0, gv0, ev0, sg0, se0)
        half(i + 1, 1, dv1, qd1, gv1, ev1, sg1, se1)
        half(i + 2, 2, dv0, qd0, gv0, ev0, sg0, se0)
        half(i + 3, 3, dv1, qd1, gv1, ev1, sg1, se1)
        return carry

    lax.fori_loop(0, NCHUNK // 4, quad, 0)
    for r, (m0, dvb, qdb, gvb, evb, sgb, seb) in enumerate((
            (0, dv0, qd0, gv0, ev0, sg0, se0),
            (1, dv1, qd1, gv1, ev1, sg1, se1))[:NCHUNK % 4]):
        half(NCHUNK - (NCHUNK % 4) + r, m0, dvb, qdb, gvb, evb, sgb, seb)

    plsc.subcore_barrier()

    @pl.when(s < NS - 1)
    def _():
        pltpu.sync_copy(aggr.at[pl.ds(s * RPT, RPT)],
                        out_hbm.at[pl.ds(c * N + s * RPT, RPT)])

    @pl.when(s == NS - 1)
    def _():
        pltpu.sync_copy(aggr.at[pl.ds(15 * RPT, RPT_LAST)],
                        out_hbm.at[pl.ds(c * N + 15 * RPT, RPT_LAST)])


_sc_aggr = pl.kernel(
    _sc_body,
    out_type=jax.ShapeDtypeStruct((2 * N, D), F32),
    mesh=plsc.VectorSubcoreMesh(core_axis_name="c", subcore_axis_name="s",
                                num_cores=NC, num_subcores=NS),
    scratch_types=(
        [pltpu.VMEM((CH,), jnp.int32)] * 6
        + [pltpu.VMEM((CH, D), F32)] * 4
        + [pltpu.VMEM_SHARED((N, D), F32)]
        + [pltpu.SemaphoreType.DMA] * 10
    ),
)


BE = 3200


def _edge_lin_body(ea_ref, w_ref, b_ref, o1, o2, o3):
    ea = ea_ref[...]
    for l, o in enumerate((o1, o2, o3)):
        o[...] = (jnp.dot(ea, w_ref[l]) + b_ref[l])


def _edge_lin(ea, w_stack, b_stack):
    eshape = jax.ShapeDtypeStruct((E, D), F32)
    return pl.pallas_call(
        _edge_lin_body,
        grid=(E // BE,),
        in_specs=[
            pl.BlockSpec((BE, DE), lambda i: (i, 0)),
            pl.BlockSpec((3, DE, D), lambda i: (0, 0, 0)),
            pl.BlockSpec((3, 1, D), lambda i: (0, 0, 0)),
        ],
        out_specs=[pl.BlockSpec((BE, D), lambda i: (i, 0))] * 3,
        out_shape=[eshape, eshape, eshape],
    )(ea, w_stack, b_stack)


BN = 2000


def _mlp_body(h_ref, p_ref, w1_ref, b1_ref, w2_ref, b2_ref, o_ref):
    y = h_ref[...] + p_ref[0] + p_ref[1]
    t = jax.nn.relu(jnp.dot(y, w1_ref[...]) + b1_ref[...])
    o_ref[...] = jax.nn.relu(
        jnp.dot(t, w2_ref[...]) + b2_ref[...])


def _mlp(h, parts, w1, b1, w2, b2):
    return pl.pallas_call(
        _mlp_body,
        grid=(N // BN,),
        in_specs=[
            pl.BlockSpec((BN, D), lambda i: (i, 0)),
            pl.BlockSpec((2, BN, D), lambda i: (0, i, 0)),
            pl.BlockSpec((D, D), lambda i: (0, 0)),
            pl.BlockSpec((1, D), lambda i: (0, 0)),
            pl.BlockSpec((D, D), lambda i: (0, 0)),
            pl.BlockSpec((1, D), lambda i: (0, 0)),
        ],
        out_specs=pl.BlockSpec((BN, D), lambda i: (i, 0)),
        out_shape=jax.ShapeDtypeStruct((N, D), F32),
    )(h, parts, w1, b1.reshape(1, D), w2, b2.reshape(1, D))


def _pool_body(h_ref, b_ref, wih_ref, bih_ref, bhh_ref, rw_ref, rb_ref, o_ref):
    h = h_ref[...]
    gids = lax.broadcasted_iota(jnp.int32, (N, G), 1)
    onehot = (b_ref[...] == gids).astype(F32)
    sums = lax.dot_general(onehot, h, (((0,), (0,)), ((), ())),
                           precision=lax.Precision.HIGHEST)
    cnts = jnp.sum(onehot, axis=0)
    pooled = sums / jnp.maximum(cnts, 1.0)[:, None]
    z = lax.dot_general(pooled, wih_ref[...], (((1,), (1,)), ((), ())))
    z = z + bih_ref[...] + bhh_ref[...]
    i_g = z[:, 0:D]
    g_g = z[:, 2 * D:3 * D]
    o_g = z[:, 3 * D:4 * D]
    cst = jax.nn.sigmoid(i_g) * jnp.tanh(g_g)
    hh = jax.nn.sigmoid(o_g) * jnp.tanh(cst)
    o_ref[...] = jnp.dot(hh, rw_ref[...]) + rb_ref[...]


def _pool_lstm(h, batch2d, wih, bih, bhh, rw, rb):
    full = lambda shape: pl.BlockSpec(shape, lambda: tuple(0 for _ in shape))
    return pl.pallas_call(
        _pool_body,
        in_specs=[
            full((N, D)), full((N, 1)), full((4 * D, D)),
            full((1, 4 * D)), full((1, 4 * D)), full((D, 1)), full((1, 1)),
        ],
        out_specs=full((G, 1)),
        out_shape=jax.ShapeDtypeStruct((G, 1), F32),
    )(h, batch2d, wih, bih.reshape(1, 4 * D), bhh.reshape(1, 4 * D),
      rw, rb.reshape(1, 1))


def kernel(x, edge_index, edge_attr, batch, params):
    p = params
    w_stack = jnp.stack([p['lin1_W'], p['lin2_W'], p['lin3_W']])
    b_stack = jnp.stack([p['lin1_b'], p['lin2_b'], p['lin3_b']])[:, None, :]
    eb1, eb2, eb3 = _edge_lin(edge_attr, w_stack, b_stack)
    zr = jnp.zeros((RPT_LAST, D), F32)
    src = edge_index[0]
    dst = edge_index[1]

    h = x
    for eb, wk in ((eb1, 'n1'), (eb2, 'n2'), (eb3, 'n3')):
        parts = _sc_aggr(h, eb, src, dst, zr)
        parts = parts.reshape(2, N, D)
        h = _mlp(h, parts, p[wk + '_W1'], p[wk + '_b1'],
                 p[wk + '_W2'], p[wk + '_b2'])

    out = _pool_lstm(h, batch.reshape(N, 1), p['Wih'], p['bih'], p['bhh'],
                     p['reg_W'], p['reg_b'])
    return out[:, 0]


# speedup vs baseline: 5.1080x; 1.0642x over previous
import functools

import jax
import jax.numpy as jnp
from jax import lax
from jax.experimental import pallas as pl
from jax.experimental.pallas import tpu as pltpu
from jax.experimental.pallas import tpu_sc as plsc

N = 10000
E = 320000
D = 128
DE = 16
G = 64

NC = 2
NS = 16
CH = 40
EPW = E // (NC * NS)
RPT = 624
RPT_LAST = N - 15 * RPT
F32 = jnp.float32


NCHUNK = EPW // CH


def _sc_body(h_hbm, eb_hbm, src_hbm, dst_hbm, zr_hbm, out_hbm,
             sv0, sv1, sv2, sv3, dv0, dv1, dv2, dv3,
             gv0, gv1, ev0, ev1, mv0, mv1, aggr,
             qs0, qs1, qs2, qs3, qd0, qd1, qd2, qd3,
             sg0, sg1, se0, se1, ss0, ss1):
    c = lax.axis_index("c")
    s = lax.axis_index("s")
    wid = s * NC + c
    base = wid * EPW
    svs = (sv0, sv1, sv2, sv3)
    qss = (qs0, qs1, qs2, qs3)
    dvs = (dv0, dv1, dv2, dv3)
    qds = (qd0, qd1, qd2, qd3)

    def sfire(m, j):
        pltpu.async_copy(src_hbm.at[pl.ds(base + j * CH, CH)], svs[m], qss[m])

    def swait(m, j):
        pltpu.make_async_copy(src_hbm.at[pl.ds(base + j * CH, CH)],
                              svs[m], qss[m]).wait()

    def dfire(m, j):
        pltpu.async_copy(dst_hbm.at[pl.ds(base + j * CH, CH)], dvs[m], qds[m])

    def dwait(m, j):
        pltpu.make_async_copy(dst_hbm.at[pl.ds(base + j * CH, CH)],
                              dvs[m], qds[m]).wait()

    def gefire(j, svb, gvb, evb, sgb, seb):
        pltpu.async_copy(h_hbm.at[svb], gvb, sgb)
        pltpu.async_copy(eb_hbm.at[pl.ds(base + j * CH, CH)], evb, seb)

    def gewait(j, svb, gvb, evb, sgb, seb):
        pltpu.make_async_copy(h_hbm.at[svb], gvb, sgb).wait()
        pltpu.make_async_copy(eb_hbm.at[pl.ds(base + j * CH, CH)], evb,
                              seb).wait()

    def compute(gvb, evb, mvb):
        @plsc.parallel_loop(0, CH, unroll=2)
        def _(r):
            for k in range(D // 16):
                sl = pl.ds(k * 16, 16)
                mvb[r, sl] = jnp.maximum(gvb[r, sl] + evb[r, sl], 0.0)

    def scfire(m, mvb, ssb):
        pltpu.async_copy(mvb, aggr.at[dvs[m]], ssb, add=True)

    def scwait(m, mvb, ssb):
        pltpu.make_async_copy(mvb, aggr.at[dvs[m]], ssb).wait()

    for m in range(4):
        sfire(m, m)
        dfire(m, m)
    swait(0, 0)
    swait(1, 1)
    gefire(0, sv0, gv0, ev0, sg0, se0)
    gefire(1, sv1, gv1, ev1, sg1, se1)

    @pl.when(s < NS - 1)
    def _():
        pltpu.sync_copy(zr_hbm.at[pl.ds(0, RPT)], aggr.at[pl.ds(s * RPT, RPT)])

    @pl.when(s == NS - 1)
    def _():
        pltpu.sync_copy(zr_hbm, aggr.at[pl.ds(15 * RPT, RPT_LAST)])

    plsc.subcore_barrier()

    def half(i, m0, gvb, evb, mvb, sgb, seb, ssb):
        m2 = (m0 + 2) % 4
        gewait(i, svs[m0], gvb, evb, sgb, seb)

        @pl.when(i + 4 < NCHUNK)
        def _():
            sfire(m0, i + 4)

        @pl.when(i >= 2)
        def _():
            scwait(m2, mvb, ssb)
            @pl.when(i + 2 < NCHUNK)
            def _():
                dfire(m2, i + 2)

        dwait(m0, i)
        compute(gvb, evb, mvb)
        scfire(m0, mvb, ssb)

        @pl.when(i + 2 < NCHUNK)
        def _():
            swait(m2, i + 2)
            gefire(i + 2, svs[m2], gvb, evb, sgb, seb)

    def quad(q, carry):
        i = 4 * q
        half(i, 0, gv0, ev0, mv0, sg0, se0, ss0)
        half(i + 1, 1, gv1, ev1, mv1, sg1, se1, ss1)
        half(i + 2, 2, gv0, ev0, mv0, sg0, se0, ss0)
        half(i + 3, 3, gv1, ev1, mv1, sg1, se1, ss1)
        return carry

    lax.fori_loop(0, NCHUNK // 4, quad, 0)
    half(NCHUNK - 2, 0, gv0, ev0, mv0, sg0, se0, ss0)
    half(NCHUNK - 1, 1, gv1, ev1, mv1, sg1, se1, ss1)
    scwait(2, mv0, ss0)
    scwait(3, mv1, ss1)

    plsc.subcore_barrier()

    @pl.when(s < NS - 1)
    def _():
        pltpu.sync_copy(aggr.at[pl.ds(s * RPT, RPT)],
                        out_hbm.at[pl.ds(c * N + s * RPT, RPT)])

    @pl.when(s == NS - 1)
    def _():
        pltpu.sync_copy(aggr.at[pl.ds(15 * RPT, RPT_LAST)],
                        out_hbm.at[pl.ds(c * N + 15 * RPT, RPT_LAST)])


_sc_aggr = pl.kernel(
    _sc_body,
    out_type=jax.ShapeDtypeStruct((2 * N, D), F32),
    mesh=plsc.VectorSubcoreMesh(core_axis_name="c", subcore_axis_name="s",
                                num_cores=NC, num_subcores=NS),
    scratch_types=(
        [pltpu.VMEM((CH,), jnp.int32)] * 8
        + [pltpu.VMEM((CH, D), F32)] * 6
        + [pltpu.VMEM_SHARED((N, D), F32)]
        + [pltpu.SemaphoreType.DMA] * 14
    ),
)


BE = 3200


def _edge_lin_body(ea_ref, w_ref, b_ref, o1, o2, o3):
    ea = ea_ref[...]
    for l, o in enumerate((o1, o2, o3)):
        o[...] = (jnp.dot(ea, w_ref[l]) + b_ref[l])


def _edge_lin(ea, w_stack, b_stack):
    eshape = jax.ShapeDtypeStruct((E, D), F32)
    return pl.pallas_call(
        _edge_lin_body,
        grid=(E // BE,),
        in_specs=[
            pl.BlockSpec((BE, DE), lambda i: (i, 0)),
            pl.BlockSpec((3, DE, D), lambda i: (0, 0, 0)),
            pl.BlockSpec((3, 1, D), lambda i: (0, 0, 0)),
        ],
        out_specs=[pl.BlockSpec((BE, D), lambda i: (i, 0))] * 3,
        out_shape=[eshape, eshape, eshape],
    )(ea, w_stack, b_stack)


BN = 2000


def _mlp_body(h_ref, p_ref, w1_ref, b1_ref, w2_ref, b2_ref, o_ref):
    y = h_ref[...] + p_ref[0] + p_ref[1]
    t = jax.nn.relu(jnp.dot(y, w1_ref[...]) + b1_ref[...])
    o_ref[...] = jax.nn.relu(
        jnp.dot(t, w2_ref[...]) + b2_ref[...])


def _mlp(h, parts, w1, b1, w2, b2):
    return pl.pallas_call(
        _mlp_body,
        grid=(N // BN,),
        in_specs=[
            pl.BlockSpec((BN, D), lambda i: (i, 0)),
            pl.BlockSpec((2, BN, D), lambda i: (0, i, 0)),
            pl.BlockSpec((D, D), lambda i: (0, 0)),
            pl.BlockSpec((1, D), lambda i: (0, 0)),
            pl.BlockSpec((D, D), lambda i: (0, 0)),
            pl.BlockSpec((1, D), lambda i: (0, 0)),
        ],
        out_specs=pl.BlockSpec((BN, D), lambda i: (i, 0)),
        out_shape=jax.ShapeDtypeStruct((N, D), F32),
    )(h, parts, w1, b1.reshape(1, D), w2, b2.reshape(1, D))


def _pool_body(h_ref, b_ref, wih_ref, bih_ref, bhh_ref, rw_ref, rb_ref, o_ref):
    h = h_ref[...]
    gids = lax.broadcasted_iota(jnp.int32, (N, G), 1)
    onehot = (b_ref[...] == gids).astype(F32)
    sums = lax.dot_general(onehot, h, (((0,), (0,)), ((), ())),
                           precision=lax.Precision.HIGHEST)
    cnts = jnp.sum(onehot, axis=0)
    pooled = sums / jnp.maximum(cnts, 1.0)[:, None]
    z = lax.dot_general(pooled, wih_ref[...], (((1,), (1,)), ((), ())))
    z = z + bih_ref[...] + bhh_ref[...]
    i_g = z[:, 0:D]
    g_g = z[:, 2 * D:3 * D]
    o_g = z[:, 3 * D:4 * D]
    cst = jax.nn.sigmoid(i_g) * jnp.tanh(g_g)
    hh = jax.nn.sigmoid(o_g) * jnp.tanh(cst)
    o_ref[...] = jnp.dot(hh, rw_ref[...]) + rb_ref[...]


def _pool_lstm(h, batch2d, wih, bih, bhh, rw, rb):
    full = lambda shape: pl.BlockSpec(shape, lambda: tuple(0 for _ in shape))
    return pl.pallas_call(
        _pool_body,
        in_specs=[
            full((N, D)), full((N, 1)), full((4 * D, D)),
            full((1, 4 * D)), full((1, 4 * D)), full((D, 1)), full((1, 1)),
        ],
        out_specs=full((G, 1)),
        out_shape=jax.ShapeDtypeStruct((G, 1), F32),
    )(h, batch2d, wih, bih.reshape(1, 4 * D), bhh.reshape(1, 4 * D),
      rw, rb.reshape(1, 1))


def kernel(x, edge_index, edge_attr, batch, params):
    p = params
    w_stack = jnp.stack([p['lin1_W'], p['lin2_W'], p['lin3_W']])
    b_stack = jnp.stack([p['lin1_b'], p['lin2_b'], p['lin3_b']])[:, None, :]
    eb1, eb2, eb3 = _edge_lin(edge_attr, w_stack, b_stack)
    zr = jnp.zeros((RPT_LAST, D), F32)
    src = edge_index[0]
    dst = edge_index[1]

    h = x
    for eb, wk in ((eb1, 'n1'), (eb2, 'n2'), (eb3, 'n3')):
        parts = _sc_aggr(h, eb, src, dst, zr)
        parts = parts.reshape(2, N, D)
        h = _mlp(h, parts, p[wk + '_W1'], p[wk + '_b1'],
                 p[wk + '_W2'], p[wk + '_b2'])

    out = _pool_lstm(h, batch.reshape(N, 1), p['Wih'], p['bih'], p['bhh'],
                     p['reg_W'], p['reg_b'])
    return out[:, 0]
